# baseline (device time: 261559 ns/iter reference)
import jax
import jax.numpy as jnp
from jax import lax
from jax.experimental import pallas as pl
from jax.experimental.pallas import tpu as pltpu

N_DEV = 8
SQ = 1024
SKV = 1024
HS = 512
H_PER = 8
DH = 128
BLK = 64
SCALE = 0.08838834764831843

FRAGS = [
    (6, 0, 3), (5, 0, 4), (1, 0, None),
    (6, 1, 3), (5, 1, 4), (1, 1, None),
    (7, 0, None), (7, 1, 4), (2, 0, None),
    (3, 0, None), (4, 0, None), (2, 1, None),
    (3, 1, None), (4, 1, None),
    (0, 0, None), (0, 1, None),
]
N_SEND = 14
RELAY_PLAN = {3: [(0, 6, 0), (1, 6, 1)], 4: [(0, 5, 0), (1, 5, 1), (2, 7, 1)]}

f32 = jnp.float32
bf16 = jnp.bfloat16


def kernel(x, Wq, K_ext, V_ext, Wo):
    def body(x_ref, wq_ref, k_ref, v_ref, wo_ref, out_ref,
             kvloc, ctxbuf, tmpk, tmpv, stage, rbuf,
             s1, r1, s2, r2, s3, r3, rg3, rg2, rg1,
             copy_sems, kv_send_sems, kv_recv_sems, rl_send_sems,
             rl_recv_sems, ar_send_sems, ar_recv_sems):
        me = lax.axis_index("i")

        descs = []
        for i, (p, half, via) in enumerate(FRAGS[:N_SEND]):
            if via is None:
                dst = kvloc.at[pl.ds(half * HS, HS)]
                rsem = kv_recv_sems.at[half]
                dev = p
            else:
                slot = [e[0] for e in RELAY_PLAN[via]
                        if e[1] == p and e[2] == half][0]
                dst = rbuf.at[slot]
                rsem = rl_recv_sems.at[slot]
                dev = via
            descs.append(pltpu.make_async_remote_copy(
                src_ref=stage.at[i % 2],
                dst_ref=dst,
                send_sem=kv_send_sems.at[i],
                recv_sem=rsem,
                device_id=(dev,),
                device_id_type=pl.DeviceIdType.MESH,
            ))

        fwd = {}
        rwait = {}
        for via, plan in RELAY_PLAN.items():
            for slot, tgt, half in plan:
                fwd[(via, slot)] = pltpu.make_async_remote_copy(
                    src_ref=rbuf.at[slot],
                    dst_ref=kvloc.at[pl.ds(half * HS, HS)],
                    send_sem=rl_send_sems.at[slot],
                    recv_sem=kv_recv_sems.at[half],
                    device_id=(tgt,),
                    device_id_type=pl.DeviceIdType.MESH,
                )
                rwait[(via, slot)] = pltpu.make_async_remote_copy(
                    src_ref=stage.at[0],
                    dst_ref=rbuf.at[slot],
                    send_sem=kv_send_sems.at[0],
                    recv_sem=rl_recv_sems.at[slot],
                    device_id=(0,),
                    device_id_type=pl.DeviceIdType.MESH,
                )
        own_waits = [
            pltpu.make_async_remote_copy(
                src_ref=stage.at[0],
                dst_ref=kvloc.at[pl.ds(half * HS, HS)],
                send_sem=kv_send_sems.at[0],
                recv_sem=kv_recv_sems.at[half],
                device_id=(0,),
                device_id_type=pl.DeviceIdType.MESH,
            )
            for half in (0, 1)
        ]

        DEPTH = 4

        def load_descs(i):
            p, half, _ = FRAGS[i]
            sl = i % DEPTH
            kc = pltpu.make_async_copy(
                k_ref.at[0, pl.ds(half * HS, HS), pl.ds(H_PER * p, H_PER), :],
                tmpk.at[sl], copy_sems.at[2 * sl])
            vc = pltpu.make_async_copy(
                v_ref.at[0, pl.ds(half * HS, HS), pl.ds(H_PER * p, H_PER), :],
                tmpv.at[sl], copy_sems.at[2 * sl + 1])
            return kc, vc

        @pl.when(me == 0)
        def _():
            for j in range(DEPTH - 1):
                for d in load_descs(j):
                    d.start()
            for i in range(len(FRAGS)):
                if i + DEPTH - 1 < len(FRAGS):
                    for d in load_descs(i + DEPTH - 1):
                        d.start()
                kc, vc = load_descs(i)
                tsl = i % DEPTH
                if i < N_SEND:
                    if i >= 2:
                        descs[i - 2].wait_send()
                    dst = stage.at[i % 2]
                else:
                    half = FRAGS[i][1]
                    dst = kvloc.at[pl.ds(half * HS, HS)]
                kc.wait()
                dst[:, pl.ds(0, H_PER), :] = tmpk[tsl].astype(bf16)
                vc.wait()
                dst[:, pl.ds(H_PER, H_PER), :] = tmpv[tsl].astype(bf16)
                if i < N_SEND:
                    descs[i].start()
            descs[-2].wait_send()
            descs[-1].wait_send()

        q = jnp.dot(x_ref[0].astype(bf16), wq_ref[:, :].astype(bf16),
                    preferred_element_type=f32)
        q3 = q.reshape(SQ, H_PER, DH).astype(bf16)

        for via, plan in RELAY_PLAN.items():
            @pl.when(me == via)
            def _(via=via, plan=plan):
                for slot, _, _ in plan:
                    rwait[(via, slot)].wait_recv()
                    fwd[(via, slot)].start()

        @pl.when(me != 0)
        def _():
            own_waits[0].wait_recv()
            own_waits[1].wait_recv()

        qb = lax.broadcasted_iota(jnp.int32, (SQ, SKV), 0) // BLK
        kb = lax.broadcasted_iota(jnp.int32, (SQ, SKV), 1) // BLK
        mask = kb <= qb
        for h in range(H_PER):
            s = lax.dot_general(
                q3[:, h, :], kvloc[:, h, :], (((1,), (1,)), ((), ())),
                preferred_element_type=f32,
            ) * SCALE
            s = jnp.where(mask, s, f32(-1e9))
            m = jnp.max(s, axis=1, keepdims=True)
            w = jnp.exp(s - m)
            w = (w / jnp.sum(w, axis=1, keepdims=True)).astype(bf16)
            ctxbuf[:, pl.ds(h * DH, DH)] = jnp.dot(
                w, kvloc[:, H_PER + h, :],
                preferred_element_type=f32).astype(bf16)

        p0 = jnp.dot(ctxbuf[:, :], wo_ref[:, :].astype(bf16),
                     preferred_element_type=f32)

        m4 = me % 4
        zb = me // 4
        yb = m4 // 2
        xb = (m4 // 2 + m4 % 2) % 2
        pz = (me + 4) % N_DEV
        py = me + 3 - 2 * m4
        px = me + 1 - 2 * (m4 % 2)

        def exchange(sbuf, rbuf_, val_bf, partner, sem_idx):
            sbuf[...] = val_bf
            d = pltpu.make_async_remote_copy(
                src_ref=sbuf, dst_ref=rbuf_,
                send_sem=ar_send_sems.at[sem_idx],
                recv_sem=ar_recv_sems.at[sem_idx],
                device_id=(partner,),
                device_id_type=pl.DeviceIdType.MESH,
            )
            d.start()
            d.wait_recv()
            return d

        half = SQ // 2
        d1 = exchange(s1, r1,
                      jnp.where(zb == 0, p0[half:], p0[:half]).astype(bf16),
                      pz, 0)
        a1 = jnp.where(zb == 0, p0[:half], p0[half:]) + r1[:, :].astype(f32)
        half //= 2
        d2 = exchange(s2, r2,
                      jnp.where(yb == 0, a1[half:], a1[:half]).astype(bf16),
                      py, 1)
        a2 = jnp.where(yb == 0, a1[:half], a1[half:]) + r2[:, :].astype(f32)
        half //= 2
        d3 = exchange(s3, r3,
                      jnp.where(xb == 0, a2[half:], a2[:half]).astype(bf16),
                      px, 2)
        a3 = jnp.where(xb == 0, a2[:half], a2[half:]) + r3[:, :].astype(f32)

        d3.wait_send()
        d4 = exchange(s3, rg3, a3.astype(bf16), px, 3)
        b2 = jnp.where(
            xb == 0,
            jnp.concatenate([s3[:, :], rg3[:, :]], axis=0),
            jnp.concatenate([rg3[:, :], s3[:, :]], axis=0))
        d2.wait_send()
        d5 = exchange(s2, rg2, b2, py, 4)
        b1 = jnp.where(
            yb == 0,
            jnp.concatenate([s2[:, :], rg2[:, :]], axis=0),
            jnp.concatenate([rg2[:, :], s2[:, :]], axis=0))
        d1.wait_send()
        d6 = exchange(s1, rg1, b1, pz, 5)
        full = jnp.where(
            zb == 0,
            jnp.concatenate([s1[:, :], rg1[:, :]], axis=0),
            jnp.concatenate([rg1[:, :], s1[:, :]], axis=0))
        out_ref[0] = full.astype(f32)

        for d in (d4, d5, d6):
            d.wait_send()

        for via, plan in RELAY_PLAN.items():
            @pl.when(me == via)
            def _(via=via, plan=plan):
                for slot, _, _ in plan:
                    fwd[(via, slot)].wait_send()

    return pl.pallas_call(
        body,
        out_shape=jax.ShapeDtypeStruct((1, SQ, SQ), f32),
        in_specs=[
            pl.BlockSpec(memory_space=pltpu.VMEM),
            pl.BlockSpec(memory_space=pltpu.VMEM),
            pl.BlockSpec(memory_space=pltpu.MemorySpace.HBM),
            pl.BlockSpec(memory_space=pltpu.MemorySpace.HBM),
            pl.BlockSpec(memory_space=pltpu.VMEM),
        ],
        out_specs=pl.BlockSpec(memory_space=pltpu.VMEM),
        scratch_shapes=[
            pltpu.VMEM((SKV, 2 * H_PER, DH), bf16),
            pltpu.VMEM((SQ, H_PER * DH), bf16),
            pltpu.VMEM((4, HS, H_PER, DH), f32),
            pltpu.VMEM((4, HS, H_PER, DH), f32),
            pltpu.VMEM((2, HS, 2 * H_PER, DH), bf16),
            pltpu.VMEM((3, HS, 2 * H_PER, DH), bf16),
            pltpu.VMEM((SQ // 2, SQ), bf16),
            pltpu.VMEM((SQ // 2, SQ), bf16),
            pltpu.VMEM((SQ // 4, SQ), bf16),
            pltpu.VMEM((SQ // 4, SQ), bf16),
            pltpu.VMEM((SQ // 8, SQ), bf16),
            pltpu.VMEM((SQ // 8, SQ), bf16),
            pltpu.VMEM((SQ // 8, SQ), bf16),
            pltpu.VMEM((SQ // 4, SQ), bf16),
            pltpu.VMEM((SQ // 2, SQ), bf16),
            pltpu.SemaphoreType.DMA((8,)),
            pltpu.SemaphoreType.DMA((N_SEND,)),
            pltpu.SemaphoreType.DMA((2,)),
            pltpu.SemaphoreType.DMA((3,)),
            pltpu.SemaphoreType.DMA((3,)),
            pltpu.SemaphoreType.DMA((6,)),
            pltpu.SemaphoreType.DMA((6,)),
        ],
        compiler_params=pltpu.CompilerParams(
            vmem_limit_bytes=128 * 1024 * 1024,
        ),
    )(x, Wq, K_ext, V_ext, Wo)


# device time: 211315 ns/iter; 1.2378x vs baseline; 1.2378x over previous
import jax
import jax.numpy as jnp
from jax import lax
from jax.experimental import pallas as pl
from jax.experimental.pallas import tpu as pltpu

N_DEV = 8
SQ = 1024
SKV = 1024
HS = 512
H_PER = 8
DH = 128
BLK = 64
SCALE = 0.08838834764831843

FRAGS = [
    (6, 0, 3), (5, 0, 4), (1, 0, None),
    (6, 1, 3), (5, 1, 4), (1, 1, None),
    (7, 0, None), (7, 1, 4), (2, 0, None),
    (3, 0, None), (4, 0, None), (2, 1, None),
    (3, 1, None), (4, 1, None),
    (0, 0, None), (0, 1, None),
]
N_SEND = 14
RELAY_PLAN = {3: [(0, 6, 0), (1, 6, 1)], 4: [(0, 5, 0), (1, 5, 1), (2, 7, 1)]}

f32 = jnp.float32
bf16 = jnp.bfloat16


def kernel(x, Wq, K_ext, V_ext, Wo):
    def body(x_ref, wq_ref, k_ref, v_ref, wo_ref, out_ref,
             kvloc, ctxbuf, tmpk, tmpv, stage, rbuf,
             s1, r1, s2, r2, s3, r3, rg3, rg2, rg1,
             copy_sems, kv_send_sems, kv_recv_sems, rl_send_sems,
             rl_recv_sems, ar_send_sems, ar_recv_sems):
        me = lax.axis_index("i")

        descs = []
        for i, (p, half, via) in enumerate(FRAGS[:N_SEND]):
            if via is None:
                dst = kvloc.at[pl.ds(half * HS, HS)]
                rsem = kv_recv_sems.at[half]
                dev = p
            else:
                slot = [e[0] for e in RELAY_PLAN[via]
                        if e[1] == p and e[2] == half][0]
                dst = rbuf.at[slot]
                rsem = rl_recv_sems.at[slot]
                dev = via
            descs.append(pltpu.make_async_remote_copy(
                src_ref=stage.at[i % 4],
                dst_ref=dst,
                send_sem=kv_send_sems.at[i],
                recv_sem=rsem,
                device_id=(dev,),
                device_id_type=pl.DeviceIdType.MESH,
            ))

        fwd = {}
        rwait = {}
        for via, plan in RELAY_PLAN.items():
            for slot, tgt, half in plan:
                fwd[(via, slot)] = pltpu.make_async_remote_copy(
                    src_ref=rbuf.at[slot],
                    dst_ref=kvloc.at[pl.ds(half * HS, HS)],
                    send_sem=rl_send_sems.at[slot],
                    recv_sem=kv_recv_sems.at[half],
                    device_id=(tgt,),
                    device_id_type=pl.DeviceIdType.MESH,
                )
                rwait[(via, slot)] = pltpu.make_async_remote_copy(
                    src_ref=stage.at[0],
                    dst_ref=rbuf.at[slot],
                    send_sem=kv_send_sems.at[0],
                    recv_sem=rl_recv_sems.at[slot],
                    device_id=(0,),
                    device_id_type=pl.DeviceIdType.MESH,
                )
        own_waits = [
            pltpu.make_async_remote_copy(
                src_ref=stage.at[0],
                dst_ref=kvloc.at[pl.ds(half * HS, HS)],
                send_sem=kv_send_sems.at[0],
                recv_sem=kv_recv_sems.at[half],
                device_id=(0,),
                device_id_type=pl.DeviceIdType.MESH,
            )
            for half in (0, 1)
        ]

        DEPTH = 2

        def load_descs(i):
            p, half, _ = FRAGS[i]
            sl = i % DEPTH
            kc = pltpu.make_async_copy(
                k_ref.at[0, pl.ds(half * HS, HS), pl.ds(H_PER * p, H_PER), :],
                tmpk.at[sl], copy_sems.at[2 * sl])
            vc = pltpu.make_async_copy(
                v_ref.at[0, pl.ds(half * HS, HS), pl.ds(H_PER * p, H_PER), :],
                tmpv.at[sl], copy_sems.at[2 * sl + 1])
            return kc, vc

        @pl.when(me == 0)
        def _():
            for j in range(DEPTH - 1):
                for d in load_descs(j):
                    d.start()
            for i in range(len(FRAGS)):
                if i + DEPTH - 1 < len(FRAGS):
                    for d in load_descs(i + DEPTH - 1):
                        d.start()
                kc, vc = load_descs(i)
                tsl = i % DEPTH
                if i < N_SEND:
                    if i >= 4:
                        descs[i - 4].wait_send()
                    dst = stage.at[i % 4]
                else:
                    half = FRAGS[i][1]
                    dst = kvloc.at[pl.ds(half * HS, HS)]
                kc.wait()
                dst[:, pl.ds(0, H_PER), :] = tmpk[tsl].astype(bf16)
                vc.wait()
                dst[:, pl.ds(H_PER, H_PER), :] = tmpv[tsl].astype(bf16)
                if i < N_SEND:
                    descs[i].start()
            for d in descs[N_SEND - 4:]:
                d.wait_send()

        q = jnp.dot(x_ref[0].astype(bf16), wq_ref[:, :].astype(bf16),
                    preferred_element_type=f32)
        q3 = q.reshape(SQ, H_PER, DH).astype(bf16)

        for via, plan in RELAY_PLAN.items():
            @pl.when(me == via)
            def _(via=via, plan=plan):
                for slot, _, _ in plan:
                    rwait[(via, slot)].wait_recv()
                    fwd[(via, slot)].start()

        @pl.when(me != 0)
        def _():
            own_waits[0].wait_recv()
            own_waits[1].wait_recv()

        qb = lax.broadcasted_iota(jnp.int32, (SQ, SKV), 0) // BLK
        kb = lax.broadcasted_iota(jnp.int32, (SQ, SKV), 1) // BLK
        mask = kb <= qb
        for h in range(H_PER):
            s = lax.dot_general(
                q3[:, h, :], kvloc[:, h, :], (((1,), (1,)), ((), ())),
                preferred_element_type=f32,
            ) * SCALE
            s = jnp.where(mask, s, f32(-1e9))
            m = jnp.max(s, axis=1, keepdims=True)
            w = jnp.exp(s - m)
            w = (w / jnp.sum(w, axis=1, keepdims=True)).astype(bf16)
            ctxbuf[:, pl.ds(h * DH, DH)] = jnp.dot(
                w, kvloc[:, H_PER + h, :],
                preferred_element_type=f32).astype(bf16)

        p0 = jnp.dot(ctxbuf[:, :], wo_ref[:, :].astype(bf16),
                     preferred_element_type=f32)

        m4 = me % 4
        zb = me // 4
        yb = m4 // 2
        xb = (m4 // 2 + m4 % 2) % 2
        pz = (me + 4) % N_DEV
        py = me + 3 - 2 * m4
        px = me + 1 - 2 * (m4 % 2)

        def exchange(sbuf, rbuf_, val_bf, partner, sem_idx):
            sbuf[...] = val_bf
            d = pltpu.make_async_remote_copy(
                src_ref=sbuf, dst_ref=rbuf_,
                send_sem=ar_send_sems.at[sem_idx],
                recv_sem=ar_recv_sems.at[sem_idx],
                device_id=(partner,),
                device_id_type=pl.DeviceIdType.MESH,
            )
            d.start()
            d.wait_recv()
            return d

        half = SQ // 2
        d1 = exchange(s1, r1,
                      jnp.where(zb == 0, p0[half:], p0[:half]).astype(bf16),
                      pz, 0)
        a1 = jnp.where(zb == 0, p0[:half], p0[half:]) + r1[:, :].astype(f32)
        half //= 2
        d2 = exchange(s2, r2,
                      jnp.where(yb == 0, a1[half:], a1[:half]).astype(bf16),
                      py, 1)
        a2 = jnp.where(yb == 0, a1[:half], a1[half:]) + r2[:, :].astype(f32)
        half //= 2
        d3 = exchange(s3, r3,
                      jnp.where(xb == 0, a2[half:], a2[:half]).astype(bf16),
                      px, 2)
        a3 = jnp.where(xb == 0, a2[:half], a2[half:]) + r3[:, :].astype(f32)

        d3.wait_send()
        d4 = exchange(s3, rg3, a3.astype(bf16), px, 3)
        b2 = jnp.where(
            xb == 0,
            jnp.concatenate([s3[:, :], rg3[:, :]], axis=0),
            jnp.concatenate([rg3[:, :], s3[:, :]], axis=0))
        d2.wait_send()
        d5 = exchange(s2, rg2, b2, py, 4)
        b1 = jnp.where(
            yb == 0,
            jnp.concatenate([s2[:, :], rg2[:, :]], axis=0),
            jnp.concatenate([rg2[:, :], s2[:, :]], axis=0))
        d1.wait_send()
        d6 = exchange(s1, rg1, b1, pz, 5)
        full = jnp.where(
            zb == 0,
            jnp.concatenate([s1[:, :], rg1[:, :]], axis=0),
            jnp.concatenate([rg1[:, :], s1[:, :]], axis=0))
        out_ref[0] = full.astype(f32)

        for d in (d4, d5, d6):
            d.wait_send()

        for via, plan in RELAY_PLAN.items():
            @pl.when(me == via)
            def _(via=via, plan=plan):
                for slot, _, _ in plan:
                    fwd[(via, slot)].wait_send()

    return pl.pallas_call(
        body,
        out_shape=jax.ShapeDtypeStruct((1, SQ, SQ), f32),
        in_specs=[
            pl.BlockSpec(memory_space=pltpu.VMEM),
            pl.BlockSpec(memory_space=pltpu.VMEM),
            pl.BlockSpec(memory_space=pltpu.MemorySpace.HBM),
            pl.BlockSpec(memory_space=pltpu.MemorySpace.HBM),
            pl.BlockSpec(memory_space=pltpu.VMEM),
        ],
        out_specs=pl.BlockSpec(memory_space=pltpu.VMEM),
        scratch_shapes=[
            pltpu.VMEM((SKV, 2 * H_PER, DH), bf16),
            pltpu.VMEM((SQ, H_PER * DH), bf16),
            pltpu.VMEM((2, HS, H_PER, DH), f32),
            pltpu.VMEM((2, HS, H_PER, DH), f32),
            pltpu.VMEM((4, HS, 2 * H_PER, DH), bf16),
            pltpu.VMEM((3, HS, 2 * H_PER, DH), bf16),
            pltpu.VMEM((SQ // 2, SQ), bf16),
            pltpu.VMEM((SQ // 2, SQ), bf16),
            pltpu.VMEM((SQ // 4, SQ), bf16),
            pltpu.VMEM((SQ // 4, SQ), bf16),
            pltpu.VMEM((SQ // 8, SQ), bf16),
            pltpu.VMEM((SQ // 8, SQ), bf16),
            pltpu.VMEM((SQ // 8, SQ), bf16),
            pltpu.VMEM((SQ // 4, SQ), bf16),
            pltpu.VMEM((SQ // 2, SQ), bf16),
            pltpu.SemaphoreType.DMA((8,)),
            pltpu.SemaphoreType.DMA((N_SEND,)),
            pltpu.SemaphoreType.DMA((2,)),
            pltpu.SemaphoreType.DMA((3,)),
            pltpu.SemaphoreType.DMA((3,)),
            pltpu.SemaphoreType.DMA((6,)),
            pltpu.SemaphoreType.DMA((6,)),
        ],
        compiler_params=pltpu.CompilerParams(
            vmem_limit_bytes=128 * 1024 * 1024,
        ),
    )(x, Wq, K_ext, V_ext, Wo)
